# baseline (speedup 1.0000x reference)
import functools

import jax
import jax.numpy as jnp
from jax import lax
from jax.experimental import pallas as pl
from jax.experimental.pallas import tpu as pltpu
from jax.experimental.pallas import tpu_sc as plsc

_H = 224
_W = 224
_L = 16
_NCOL = _W // _L
_NWORK = 32


def _sc_corner(x_hbm, o_hbm, buf, isem0, isem1, osem0, osem1):
    wid = lax.axis_index("s") * 2 + lax.axis_index("c")
    n_img = x_hbm.shape[0] // _NWORK
    neg16 = jnp.full((_L,), -jnp.inf, jnp.float32)
    isems = (isem0, isem1)
    osems = (osem0, osem1)

    idx15 = jnp.full((_L,), _L - 1, jnp.int32)

    def make_row_body(slot):
        def row_body(hh, hcs):
            h = hh * 2
            cw0 = neg16
            cw1 = neg16
            out_hcs = []
            for j in range(_NCOL):
                v0 = buf[slot, h, pl.ds(j * _L, _L)]
                v1 = buf[slot, h + 1, pl.ds(j * _L, _L)]
                hc0 = jnp.maximum(hcs[j], v0)
                hc1 = jnp.maximum(hc0, v1)
                s0 = jnp.maximum(plsc.cummax(hc0), cw0)
                s1 = jnp.maximum(plsc.cummax(hc1), cw1)
                cw0 = s0.at[idx15].get(mode="promise_in_bounds")
                cw1 = s1.at[idx15].get(mode="promise_in_bounds")
                buf[slot, h, pl.ds(j * _L, _L)] = s0 + s0
                buf[slot, h + 1, pl.ds(j * _L, _L)] = s1 + s1
                out_hcs.append(hc1)
            return tuple(out_hcs)
        return row_body

    base = wid * n_img
    in_h = {}
    out_h = {}
    in_h[0] = pltpu.async_copy(x_hbm.at[base], buf.at[0], isems[0])
    for i in range(n_img):
        s = i % 2
        if i + 1 < n_img:
            if i >= 1:
                out_h[i - 1].wait()
            in_h[i + 1] = pltpu.async_copy(x_hbm.at[base + i + 1],
                                           buf.at[(i + 1) % 2],
                                           isems[(i + 1) % 2])
        in_h[i].wait()
        lax.fori_loop(0, _H // 2, make_row_body(s), tuple([neg16] * _NCOL))
        out_h[i] = pltpu.async_copy(buf.at[s], o_hbm.at[base + i], osems[s])
    out_h[n_img - 2].wait()
    out_h[n_img - 1].wait()


def kernel(x):
    b, c, h, w = x.shape
    xf = x.reshape(b * c, h, w)
    fn = functools.partial(
        pl.kernel,
        mesh=plsc.VectorSubcoreMesh(core_axis_name="c", subcore_axis_name="s"),
        out_type=jax.ShapeDtypeStruct((b * c, h, w), jnp.float32),
        scratch_types=[
            pltpu.VMEM((2, h, w), jnp.float32),
            pltpu.SemaphoreType.DMA,
            pltpu.SemaphoreType.DMA,
            pltpu.SemaphoreType.DMA,
            pltpu.SemaphoreType.DMA,
        ],
        compiler_params=pltpu.CompilerParams(needs_layout_passes=False),
    )(_sc_corner)
    return fn(xf).reshape(b, c, h, w)


# speedup vs baseline: 1.0626x; 1.0626x over previous
import functools

import jax
import jax.numpy as jnp
from jax import lax
from jax.experimental import pallas as pl
from jax.experimental.pallas import tpu as pltpu
from jax.experimental.pallas import tpu_sc as plsc

_H = 224
_W = 224
_L = 16
_NCOL = _W // _L
_NWORK = 32
_NSLOT = 4
_RC = _H // 2


def _sc_corner(x_hbm, o_hbm, buf, *sems):
    wid = lax.axis_index("s") * 2 + lax.axis_index("c")
    n_img = x_hbm.shape[0] // _NWORK
    n_chunk = n_img * 2
    neg16 = jnp.full((_L,), -jnp.inf, jnp.float32)
    isems = sems[:_NSLOT]
    osems = sems[_NSLOT:]
    base = wid * n_img

    def start_in(k):
        img = base + k // 2
        r0 = (k % 2) * _RC
        return pltpu.async_copy(x_hbm.at[img, pl.ds(r0, _RC)],
                                buf.at[k % _NSLOT], isems[k % _NSLOT])

    def start_out(k):
        img = base + k // 2
        r0 = (k % 2) * _RC
        return pltpu.async_copy(buf.at[k % _NSLOT],
                                o_hbm.at[img, pl.ds(r0, _RC)],
                                osems[k % _NSLOT])

    def make_row_body(slot):
        def row_body(h, hcs):
            cw = neg16
            out_hcs = []
            for j in range(_NCOL):
                v = buf[slot, h, pl.ds(j * _L, _L)]
                hc = jnp.maximum(hcs[j], v)
                s = jnp.maximum(plsc.cummax(hc), cw)
                cw = jnp.maximum(cw, jnp.full((_L,),
                                              lax.reduce_max(hc, (0,))))
                buf[slot, h, pl.ds(j * _L, _L)] = s + s
                out_hcs.append(hc)
            return tuple(out_hcs)
        return row_body

    in_h = {0: start_in(0), 1: start_in(1)}
    out_h = {}
    hcs = tuple([neg16] * _NCOL)
    for k in range(n_chunk):
        s = k % _NSLOT
        if k + 2 < n_chunk:
            if k - 2 >= 0:
                out_h[k - 2].wait()
            in_h[k + 2] = start_in(k + 2)
        in_h[k].wait()
        if k % 2 == 0:
            hcs = tuple([neg16] * _NCOL)
        hcs = lax.fori_loop(0, _RC, make_row_body(s), hcs)
        out_h[k] = start_out(k)
    for k in range(n_chunk - 4, n_chunk):
        out_h[k].wait()


def kernel(x):
    b, c, h, w = x.shape
    xf = x.reshape(b * c, h, w)
    fn = functools.partial(
        pl.kernel,
        mesh=plsc.VectorSubcoreMesh(core_axis_name="c", subcore_axis_name="s"),
        out_type=jax.ShapeDtypeStruct((b * c, h, w), jnp.float32),
        scratch_types=(
            [pltpu.VMEM((_NSLOT, _RC, w), jnp.float32)]
            + [pltpu.SemaphoreType.DMA] * (2 * _NSLOT)
        ),
        compiler_params=pltpu.CompilerParams(needs_layout_passes=False),
    )(_sc_corner)
    return fn(xf).reshape(b, c, h, w)


# speedup vs baseline: 1.1160x; 1.0502x over previous
import functools

import jax
import jax.numpy as jnp
from jax import lax
from jax.experimental import pallas as pl
from jax.experimental.pallas import tpu as pltpu
from jax.experimental.pallas import tpu_sc as plsc

_H = 224
_W = 224
_L = 16
_NCOL = _W // _L
_NWORK = 32


def _sc_corner(x_hbm, o_hbm, buf, isem0, isem1, osem0, osem1):
    wid = lax.axis_index("s") * 2 + lax.axis_index("c")
    n_img = x_hbm.shape[0] // _NWORK
    neg16 = jnp.full((_L,), -jnp.inf, jnp.float32)
    isems = (isem0, isem1)
    osems = (osem0, osem1)

    def make_row_body(slot):
        def row_body(h, hcs):
            cw = neg16
            out_hcs = []
            for j in range(_NCOL):
                v = buf[slot, h, pl.ds(j * _L, _L)]
                hc = jnp.maximum(hcs[j], v)
                s = jnp.maximum(plsc.cummax(hc), cw)
                cw = jnp.maximum(cw, jnp.full((_L,),
                                              lax.reduce_max(hc, (0,))))
                buf[slot, h, pl.ds(j * _L, _L)] = s + s
                out_hcs.append(hc)
            return tuple(out_hcs)
        return row_body

    base = wid * n_img
    in_h = {}
    out_h = {}
    in_h[0] = pltpu.async_copy(x_hbm.at[base], buf.at[0], isems[0])
    for i in range(n_img):
        s = i % 2
        if i + 1 < n_img:
            if i >= 1:
                out_h[i - 1].wait()
            in_h[i + 1] = pltpu.async_copy(x_hbm.at[base + i + 1],
                                           buf.at[(i + 1) % 2],
                                           isems[(i + 1) % 2])
        in_h[i].wait()
        lax.fori_loop(0, _H, make_row_body(s), tuple([neg16] * _NCOL))
        out_h[i] = pltpu.async_copy(buf.at[s], o_hbm.at[base + i], osems[s])
    out_h[n_img - 2].wait()
    out_h[n_img - 1].wait()


def kernel(x):
    b, c, h, w = x.shape
    xf = x.reshape(b * c, h, w)
    fn = functools.partial(
        pl.kernel,
        mesh=plsc.VectorSubcoreMesh(core_axis_name="c", subcore_axis_name="s"),
        out_type=jax.ShapeDtypeStruct((b * c, h, w), jnp.float32),
        scratch_types=[
            pltpu.VMEM((2, h, w), jnp.float32),
            pltpu.SemaphoreType.DMA,
            pltpu.SemaphoreType.DMA,
            pltpu.SemaphoreType.DMA,
            pltpu.SemaphoreType.DMA,
        ],
        compiler_params=pltpu.CompilerParams(needs_layout_passes=False),
    )(_sc_corner)
    return fn(xf).reshape(b, c, h, w)


# speedup vs baseline: 1.1311x; 1.0135x over previous
import functools

import jax
import jax.numpy as jnp
from jax import lax
from jax.experimental import pallas as pl
from jax.experimental.pallas import tpu as pltpu
from jax.experimental.pallas import tpu_sc as plsc

_H = 224
_W = 224
_L = 16
_NCOL = _W // _L
_NWORK = 32


def _sc_corner(x_hbm, o_hbm, buf, isem0, isem1, osem0, osem1):
    wid = lax.axis_index("s") * 2 + lax.axis_index("c")
    n_img = x_hbm.shape[0] // _NWORK
    neg16 = jnp.full((_L,), -jnp.inf, jnp.float32)
    isems = (isem0, isem1)
    osems = (osem0, osem1)

    def make_row_body(slot):
        def row_body(h, hcs):
            cw = neg16
            out_hcs = []
            for j in range(_NCOL):
                v = buf[slot, h, pl.ds(j * _L, _L)]
                hc = jnp.maximum(hcs[j], v)
                s = jnp.maximum(plsc.cummax(hc), cw)
                cw = jnp.maximum(cw, jnp.full((_L,),
                                              lax.reduce_max(hc, (0,))))
                buf[slot, h, pl.ds(j * _L, _L)] = s + s
                out_hcs.append(hc)
            return tuple(out_hcs)
        return row_body

    base = wid * n_img
    in_h = {}
    out_h = {}
    in_h[0] = pltpu.async_copy(x_hbm.at[base], buf.at[0], isems[0])
    for i in range(n_img):
        s = i % 2
        if i + 1 < n_img:
            if i >= 1:
                out_h[i - 1][0].wait()
                out_h[i - 1][1].wait()
            in_h[i + 1] = pltpu.async_copy(x_hbm.at[base + i + 1],
                                           buf.at[(i + 1) % 2],
                                           isems[(i + 1) % 2])
        in_h[i].wait()
        hcs = lax.fori_loop(0, _H // 2, make_row_body(s),
                            tuple([neg16] * _NCOL))
        top = pltpu.async_copy(buf.at[s, pl.ds(0, _H // 2)],
                               o_hbm.at[base + i, pl.ds(0, _H // 2)],
                               osems[s])
        lax.fori_loop(_H // 2, _H, make_row_body(s), hcs)
        bot = pltpu.async_copy(buf.at[s, pl.ds(_H // 2, _H // 2)],
                               o_hbm.at[base + i, pl.ds(_H // 2, _H // 2)],
                               osems[s])
        out_h[i] = (top, bot)
    for i in (n_img - 2, n_img - 1):
        out_h[i][0].wait()
        out_h[i][1].wait()


def kernel(x):
    b, c, h, w = x.shape
    xf = x.reshape(b * c, h, w)
    fn = functools.partial(
        pl.kernel,
        mesh=plsc.VectorSubcoreMesh(core_axis_name="c", subcore_axis_name="s"),
        out_type=jax.ShapeDtypeStruct((b * c, h, w), jnp.float32),
        scratch_types=[
            pltpu.VMEM((2, h, w), jnp.float32),
            pltpu.SemaphoreType.DMA,
            pltpu.SemaphoreType.DMA,
            pltpu.SemaphoreType.DMA,
            pltpu.SemaphoreType.DMA,
        ],
        compiler_params=pltpu.CompilerParams(needs_layout_passes=False),
    )(_sc_corner)
    return fn(xf).reshape(b, c, h, w)
